# baseline (speedup 1.0000x reference)
import functools
import jax
import jax.numpy as jnp
from jax import lax
from jax.experimental import pallas as pl
from jax.experimental.pallas import tpu as pltpu
from jax.experimental.pallas import tpu_sc as plsc


def kernel(x, pos_table):
    S, B, E = x.shape
    NC, NS, L = 2, 16, 16
    NW = NC * NS
    ROWS = S * B
    RPW = ROWS // NW
    R = 128
    CH = RPW // R
    CE = R * E

    xf = x.reshape(ROWS * E)
    pf = pos_table.reshape(S * E)
    mesh = plsc.VectorSubcoreMesh(core_axis_name="c", subcore_axis_name="s")

    @functools.partial(
        pl.kernel,
        mesh=mesh,
        out_type=jax.ShapeDtypeStruct((ROWS * E,), jnp.float32),
        scratch_types=[
            pltpu.VMEM((S * E,), jnp.float32),
            pltpu.VMEM((CE,), jnp.float32),
            pltpu.VMEM((CE,), jnp.float32),
            pltpu.SemaphoreType.DMA,
            pltpu.SemaphoreType.DMA,
            pltpu.SemaphoreType.DMA,
            pltpu.SemaphoreType.DMA,
        ],
    )
    def sc_k(x_hbm, pos_hbm, out_hbm, pos_v, buf0, buf1, li0, li1, so0, so1):
        wid = lax.axis_index("s") * NC + lax.axis_index("c")
        base = wid * (RPW * E)

        pltpu.sync_copy(pos_hbm, pos_v)

        bufs = (buf0, buf1)
        lsems = (li0, li1)
        ssems = (so0, so1)

        def start_load(c, p):
            pltpu.async_copy(
                x_hbm.at[pl.ds(base + c * CE, CE)], bufs[p], lsems[p])

        def wait_load(p):
            pltpu.make_async_copy(
                x_hbm.at[pl.ds(0, CE)], bufs[p], lsems[p]).wait()

        def start_store(c, p):
            pltpu.async_copy(
                bufs[p], out_hbm.at[pl.ds(base + c * CE, CE)], ssems[p])

        def wait_store(p):
            pltpu.make_async_copy(
                bufs[p], out_hbm.at[pl.ds(0, CE)], ssems[p]).wait()

        def compute(c, p):
            buf = bufs[p]
            s = (wid * RPW + c * R) // B
            prow = [pos_v[pl.ds(s * E + j * L, L)] for j in range(E // L)]

            def row_body(r, _):
                o = r * E
                for j in range(E // L):
                    sl = pl.ds(o + j * L, L)
                    buf[sl] = buf[sl] + prow[j]
                return 0

            lax.fori_loop(0, R, row_body, 0, unroll=2)

        start_load(0, 0)

        def g_body(g, _):
            c0 = 2 * g
            c1 = 2 * g + 1
            @pl.when(g > 0)
            def _():
                wait_store(1)
            start_load(c1, 1)
            wait_load(0)
            compute(c0, 0)
            start_store(c0, 0)
            wait_load(1)
            compute(c1, 1)
            start_store(c1, 1)
            wait_store(0)
            @pl.when(g < CH // 2 - 1)
            def _():
                start_load(c0 + 2, 0)
            return 0

        lax.fori_loop(0, CH // 2, g_body, 0)
        wait_store(1)

    out = sc_k(xf, pf)
    return out.reshape(S, B, E)


# speedup vs baseline: 1.5728x; 1.5728x over previous
import jax
import jax.numpy as jnp
from jax.experimental import pallas as pl


def _add_body(x_ref, pos_ref, out_ref):
    out_ref[...] = x_ref[...] + pos_ref[...][:, None, :]


def kernel(x, pos_table):
    S, B, E = x.shape
    BS = 8
    grid = (S // BS,)
    return pl.pallas_call(
        _add_body,
        grid=grid,
        in_specs=[
            pl.BlockSpec((BS, B, E), lambda i: (i, 0, 0)),
            pl.BlockSpec((BS, E), lambda i: (i, 0)),
        ],
        out_specs=pl.BlockSpec((BS, B, E), lambda i: (i, 0, 0)),
        out_shape=jax.ShapeDtypeStruct((S, B, E), x.dtype),
    )(x, pos_table)
